# Initial kernel scaffold; baseline (speedup 1.0000x reference)
import jax
import jax.numpy as jnp
from jax.experimental import pallas as pl


def kernel(x, edge_index, edge_attr, W_node, W1, b1, W2, b2, Wr, br):
    raise NotImplementedError("write your pallas kernel here")


# speedup vs baseline: 5.3830x; 5.3830x over previous
import functools

import jax
import jax.numpy as jnp
from jax import lax
from jax.experimental import pallas as pl
from jax.experimental.pallas import tpu as pltpu
from jax.experimental.pallas import tpu_sc as plsc

N = 10000
E = 320000
CH = 128
ECH = 16
NC, NS, L = 2, 16, 16
NW = NC * NS
EPT = E // NW
C = 80
G = EPT // C
RPT = 624

ROW_BLK = 1000
CROW = 1024
EDGE_BLK = 2000


def _node_linear_body(x_ref, w_ref, y_ref):
    y_ref[...] = jnp.dot(x_ref[...], w_ref[...], preferred_element_type=jnp.float32)


def _gate_body(ea_ref, w1_ref, b1_ref, w2_ref, b2_ref, g_ref):
    h = jnp.maximum(
        jnp.dot(ea_ref[...], w1_ref[...], preferred_element_type=jnp.float32)
        + b1_ref[...], 0.0)
    z = jnp.dot(h, w2_ref[...], preferred_element_type=jnp.float32) + b2_ref[...]
    g_ref[...] = jax.nn.sigmoid(z)


def _combine_body(acc0_ref, acc1_ref, deg0_ref, deg1_ref, y_ref, x_ref,
                  wr_ref, br_ref, b1_ref, w2_ref, b2_ref, out_ref):
    i = pl.program_id(0)
    gate_loop = jax.nn.sigmoid(
        jnp.dot(jnp.maximum(b1_ref[...], 0.0), w2_ref[...],
                preferred_element_type=jnp.float32) + b2_ref[...])
    acc = acc0_ref[...] + acc1_ref[...] + y_ref[...] * gate_loop
    rows = i * CROW + lax.broadcasted_iota(jnp.int32, (CROW, CH), 0)
    acc = jnp.where(rows < N, acc, 0.0)
    invd = 1.0 / (deg0_ref[...] + deg1_ref[...] + 1.0)
    eye = (lax.broadcasted_iota(jnp.int32, (CROW, CROW), 0)
           == lax.broadcasted_iota(jnp.int32, (CROW, CROW), 1))
    dmat = jnp.where(eye, invd, 0.0)
    r = jnp.dot(x_ref[...], wr_ref[...], preferred_element_type=jnp.float32)
    out_ref[...] = (jnp.dot(dmat, acc, preferred_element_type=jnp.float32)
                    + r + br_ref[...])


def _sc_scatter_body(row_hbm, col_hbm, y_hbm, gate_hbm, acc_out, deg_out,
                     acc_sh, deg_sh, row_v, col_v, ybuf, gbuf, col_stage,
                     ones_b, dbuf, gsem):
    c = lax.axis_index("c")
    s = lax.axis_index("s")
    wid = c * NS + s
    base = wid * EPT

    zero16 = jnp.zeros((L,), jnp.float32)
    one16 = jnp.ones((L,), jnp.float32)

    def _zb(r, _):
        for j in range(CH // L):
            ybuf[r, pl.ds(j * L, L)] = zero16
        return 0
    lax.fori_loop(0, C, _zb, 0)

    for j in range(C // L):
        ones_b[pl.ds(j * L, L)] = one16

    def _zd(r, _):
        dbuf[pl.ds(r * L, L)] = zero16
        return 0
    lax.fori_loop(0, RPT // L, _zd, 0)

    r0 = s * RPT
    for j in range(7):
        pltpu.sync_copy(ybuf, acc_sh.at[pl.ds(r0 + j * C, C)])
    pltpu.sync_copy(ybuf.at[pl.ds(0, 64)], acc_sh.at[pl.ds(r0 + 7 * C, 64)])
    pltpu.sync_copy(dbuf, deg_sh.at[pl.ds(r0, RPT)])

    @pl.when(s == NS - 1)
    def _zero_tail():
        pltpu.sync_copy(ybuf.at[pl.ds(0, 16)], acc_sh.at[pl.ds(NS * RPT, 16)])
        pltpu.sync_copy(dbuf.at[pl.ds(0, L)], deg_sh.at[pl.ds(NS * RPT, L)])

    plsc.subcore_barrier()

    SCH = 2000

    def super_body(sc_i, _):
        sbase = base + sc_i * SCH
        pltpu.sync_copy(row_hbm.at[pl.ds(sbase, SCH)], row_v)
        pltpu.sync_copy(col_hbm.at[pl.ds(sbase, SCH)], col_v)

        def chunk_body(k, _):
            e0 = k * C
            for j in range(C // L):
                col_stage[pl.ds(j * L, L)] = col_v[pl.ds(e0 + j * L, L)]
            pltpu.async_copy(y_hbm.at[row_v.at[pl.ds(e0, C)]], ybuf, gsem).wait()
            pltpu.sync_copy(gate_hbm.at[pl.ds(sbase + e0, C)], gbuf)

            def mul_row(r, _):
                for j in range(CH // L):
                    sl = pl.ds(j * L, L)
                    ybuf[r, sl] = ybuf[r, sl] * gbuf[r, sl]
                return 0
            lax.fori_loop(0, C, mul_row, 0)

            pltpu.sync_copy(ybuf, acc_sh.at[col_stage], add=True)
            pltpu.sync_copy(ones_b, deg_sh.at[col_stage], add=True)
            return 0

        lax.fori_loop(0, G // 5, chunk_body, 0)
        return 0

    lax.fori_loop(0, 5, super_body, 0)

    plsc.subcore_barrier()

    for j in range(7):
        pltpu.sync_copy(acc_sh.at[pl.ds(r0 + j * C, C)], ybuf)
        pltpu.sync_copy(ybuf, acc_out.at[c, pl.ds(r0 + j * C, C)])
    pltpu.sync_copy(acc_sh.at[pl.ds(r0 + 7 * C, 64)], ybuf.at[pl.ds(0, 64)])
    pltpu.sync_copy(ybuf.at[pl.ds(0, 64)], acc_out.at[c, pl.ds(r0 + 7 * C, 64)])
    pltpu.sync_copy(deg_sh.at[pl.ds(r0, RPT)], dbuf)
    pltpu.sync_copy(dbuf, deg_out.at[pl.ds(c * N + r0, RPT)])

    @pl.when(s == NS - 1)
    def _copy_tail():
        pltpu.sync_copy(acc_sh.at[pl.ds(NS * RPT, 16)], ybuf.at[pl.ds(0, 16)])
        pltpu.sync_copy(ybuf.at[pl.ds(0, 16)], acc_out.at[c, pl.ds(NS * RPT, 16)])
        pltpu.sync_copy(deg_sh.at[pl.ds(NS * RPT, L)], dbuf.at[pl.ds(0, L)])
        pltpu.sync_copy(dbuf.at[pl.ds(0, L)], deg_out.at[pl.ds(c * N + NS * RPT, L)])


_sc_scatter = functools.partial(
    pl.kernel,
    out_type=[
        jax.ShapeDtypeStruct((NC, N, CH), jnp.float32),
        jax.ShapeDtypeStruct((NC * N,), jnp.float32),
    ],
    mesh=plsc.VectorSubcoreMesh(core_axis_name="c", subcore_axis_name="s",
                                num_cores=NC, num_subcores=NS),
    scratch_types=[
        pltpu.VMEM_SHARED((N, CH), jnp.float32),
        pltpu.VMEM_SHARED((N,), jnp.float32),
        pltpu.VMEM((2000,), jnp.int32),
        pltpu.VMEM((2000,), jnp.int32),
        pltpu.VMEM((C, CH), jnp.float32),
        pltpu.VMEM((C, CH), jnp.float32),
        pltpu.VMEM((C,), jnp.int32),
        pltpu.VMEM((C,), jnp.float32),
        pltpu.VMEM((RPT,), jnp.float32),
        pltpu.SemaphoreType.DMA,
    ],
)(_sc_scatter_body)


def kernel(x, edge_index, edge_attr, W_node, W1, b1, W2, b2, Wr, br):
    ei = edge_index.astype(jnp.int32)
    row = ei[0]
    col = ei[1]
    b1r = b1.reshape(1, CH).astype(jnp.float32)
    b2r = b2.reshape(1, CH).astype(jnp.float32)
    brr = br.reshape(1, CH).astype(jnp.float32)

    y = pl.pallas_call(
        _node_linear_body,
        grid=(N // ROW_BLK,),
        in_specs=[
            pl.BlockSpec((ROW_BLK, CH), lambda i: (i, 0)),
            pl.BlockSpec((CH, CH), lambda i: (0, 0)),
        ],
        out_specs=pl.BlockSpec((ROW_BLK, CH), lambda i: (i, 0)),
        out_shape=jax.ShapeDtypeStruct((N, CH), jnp.float32),
    )(x, W_node)

    gate = pl.pallas_call(
        _gate_body,
        grid=(E // EDGE_BLK,),
        in_specs=[
            pl.BlockSpec((EDGE_BLK, ECH), lambda i: (i, 0)),
            pl.BlockSpec((ECH, CH), lambda i: (0, 0)),
            pl.BlockSpec((1, CH), lambda i: (0, 0)),
            pl.BlockSpec((CH, CH), lambda i: (0, 0)),
            pl.BlockSpec((1, CH), lambda i: (0, 0)),
        ],
        out_specs=pl.BlockSpec((EDGE_BLK, CH), lambda i: (i, 0)),
        out_shape=jax.ShapeDtypeStruct((E, CH), jnp.float32),
    )(edge_attr, W1, b1r, W2, b2r)

    acc_parts, deg_flat = _sc_scatter(row, col, y, gate)
    deg2 = deg_flat.reshape(NC, N)

    out = pl.pallas_call(
        _combine_body,
        grid=(pl.cdiv(N, CROW),),
        in_specs=[
            pl.BlockSpec((CROW, CH), lambda i: (i, 0)),
            pl.BlockSpec((CROW, CH), lambda i: (i, 0)),
            pl.BlockSpec((1, CROW), lambda i: (0, i)),
            pl.BlockSpec((1, CROW), lambda i: (0, i)),
            pl.BlockSpec((CROW, CH), lambda i: (i, 0)),
            pl.BlockSpec((CROW, CH), lambda i: (i, 0)),
            pl.BlockSpec((CH, CH), lambda i: (0, 0)),
            pl.BlockSpec((1, CH), lambda i: (0, 0)),
            pl.BlockSpec((1, CH), lambda i: (0, 0)),
            pl.BlockSpec((CH, CH), lambda i: (0, 0)),
            pl.BlockSpec((1, CH), lambda i: (0, 0)),
        ],
        out_specs=pl.BlockSpec((CROW, CH), lambda i: (i, 0)),
        out_shape=jax.ShapeDtypeStruct((N, CH), jnp.float32),
    )(acc_parts[0], acc_parts[1], deg2[0:1], deg2[1:2], y, x,
      Wr, brr, b1r, W2, b2r)

    return out


# speedup vs baseline: 7.9398x; 1.4750x over previous
import functools

import jax
import jax.numpy as jnp
from jax import lax
from jax.experimental import pallas as pl
from jax.experimental.pallas import tpu as pltpu
from jax.experimental.pallas import tpu_sc as plsc

N = 10000
E = 320000
CH = 128
ECH = 16
NC, NS, L = 2, 16, 16
NW = NC * NS
EPT = E // NW
C = 80
G = EPT // C
RPT = 624

ROW_BLK = 1000
CROW = 1024
EDGE_BLK = 2000


def _node_linear_body(x_ref, w_ref, y_ref):
    y_ref[...] = jnp.dot(x_ref[...], w_ref[...], preferred_element_type=jnp.float32)


def _gate_body(ea_ref, w1_ref, b1_ref, w2_ref, b2_ref, g_ref):
    h = jnp.maximum(
        jnp.dot(ea_ref[...], w1_ref[...], preferred_element_type=jnp.float32)
        + b1_ref[...], 0.0)
    z = jnp.dot(h, w2_ref[...], preferred_element_type=jnp.float32) + b2_ref[...]
    g_ref[...] = jax.nn.sigmoid(z)


def _combine_body(acc0_ref, acc1_ref, deg0_ref, deg1_ref, y_ref, x_ref,
                  wr_ref, br_ref, b1_ref, w2_ref, b2_ref, out_ref):
    i = pl.program_id(0)
    gate_loop = jax.nn.sigmoid(
        jnp.dot(jnp.maximum(b1_ref[...], 0.0), w2_ref[...],
                preferred_element_type=jnp.float32) + b2_ref[...])
    acc = acc0_ref[...] + acc1_ref[...] + y_ref[...] * gate_loop
    rows = i * CROW + lax.broadcasted_iota(jnp.int32, (CROW, CH), 0)
    acc = jnp.where(rows < N, acc, 0.0)
    invd = 1.0 / (deg0_ref[...] + deg1_ref[...] + 1.0)
    eye = (lax.broadcasted_iota(jnp.int32, (CROW, CROW), 0)
           == lax.broadcasted_iota(jnp.int32, (CROW, CROW), 1))
    dmat = jnp.where(eye, invd, 0.0)
    r = jnp.dot(x_ref[...], wr_ref[...], preferred_element_type=jnp.float32)
    out_ref[...] = (jnp.dot(dmat, acc, preferred_element_type=jnp.float32)
                    + r + br_ref[...])


def _sc_scatter_body(row_hbm, col_hbm, y_hbm, gate_hbm, acc_out, deg_out,
                     acc_sh, deg_sh, row_c, col_c, ybufs, gbufs,
                     ones_b, dbuf, isem0, isem1, isem2, dsem0, dsem1,
                     asem0, asem1, gsem0, gsem1):
    c = lax.axis_index("c")
    s = lax.axis_index("s")
    wid = c * NS + s
    base = wid * EPT
    isem = (isem0, isem1, isem2)
    dsem = (dsem0, dsem1)
    asem = (asem0, asem1)
    gsem = (gsem0, gsem1)

    zero16 = jnp.zeros((L,), jnp.float32)
    one16 = jnp.ones((L,), jnp.float32)

    def _zb(r, _):
        for j in range(CH // L):
            ybufs[0, r, pl.ds(j * L, L)] = zero16
        return 0
    lax.fori_loop(0, C, _zb, 0)
    ybuf = ybufs.at[0]

    for j in range(C // L):
        ones_b[pl.ds(j * L, L)] = one16

    def _zd(r, _):
        dbuf[pl.ds(r * L, L)] = zero16
        return 0
    lax.fori_loop(0, RPT // L, _zd, 0)

    r0 = s * RPT
    for j in range(7):
        pltpu.sync_copy(ybuf, acc_sh.at[pl.ds(r0 + j * C, C)])
    pltpu.sync_copy(ybuf.at[pl.ds(0, 64)], acc_sh.at[pl.ds(r0 + 7 * C, 64)])
    pltpu.sync_copy(dbuf, deg_sh.at[pl.ds(r0, RPT)])

    @pl.when(s == NS - 1)
    def _zero_tail():
        pltpu.sync_copy(ybuf.at[pl.ds(0, 16)], acc_sh.at[pl.ds(NS * RPT, 16)])
        pltpu.sync_copy(dbuf.at[pl.ds(0, L)], deg_sh.at[pl.ds(NS * RPT, L)])

    plsc.subcore_barrier()

    def issue_idx(k, jp):
        ib = jp % 3
        e0 = base + k * C
        pltpu.async_copy(row_hbm.at[pl.ds(e0, C)], row_c.at[ib], isem[ib])
        pltpu.async_copy(col_hbm.at[pl.ds(e0, C)], col_c.at[ib], isem[ib])

    def wait_idx(k, jp):
        ib = jp % 3
        e0 = base + k * C
        pltpu.make_async_copy(row_hbm.at[pl.ds(e0, C)], row_c.at[ib], isem[ib]).wait()
        pltpu.make_async_copy(col_hbm.at[pl.ds(e0, C)], col_c.at[ib], isem[ib]).wait()

    def issue_data(k, jp):
        db, ib = jp % 2, jp % 3
        pltpu.async_copy(y_hbm.at[row_c.at[ib]], ybufs.at[db], dsem[db])
        pltpu.async_copy(gate_hbm.at[pl.ds(base + k * C, C)], gbufs.at[db], dsem[db])

    def wait_data(k, jp):
        db, ib = jp % 2, jp % 3
        pltpu.make_async_copy(y_hbm.at[row_c.at[ib]], ybufs.at[db], dsem[db]).wait()
        pltpu.make_async_copy(gate_hbm.at[pl.ds(base + k * C, C)], gbufs.at[db],
                              dsem[db]).wait()

    def issue_scat(jp):
        db, ib = jp % 2, jp % 3
        pltpu.async_copy(ybufs.at[db], acc_sh.at[col_c.at[ib]], asem[db], add=True)
        pltpu.async_copy(ones_b, deg_sh.at[col_c.at[ib]], gsem[db], add=True)

    def wait_scat(jp):
        db, ib = jp % 2, jp % 3
        pltpu.make_async_copy(ybufs.at[db], acc_sh.at[col_c.at[ib]], asem[db]).wait()
        pltpu.make_async_copy(ones_b, deg_sh.at[col_c.at[ib]], gsem[db]).wait()

    def compute(jp):
        db = jp % 2

        def mul_row(r, _):
            for q in range(CH // L):
                sl = pl.ds(q * L, L)
                ybufs[db, r, sl] = ybufs[db, r, sl] * gbufs[db, r, sl]
            return 0
        lax.fori_loop(0, C, mul_row, 0)

    def pipe_iter(k, jp, first=False, with_next=True, with_next2=True):
        if not first:
            wait_scat(jp - 1)
        if with_next:
            wait_idx(k + 1, jp + 1)
            issue_data(k + 1, jp + 1)
        if with_next2:
            issue_idx(k + 2, jp + 2)
        wait_data(k, jp)
        compute(jp)
        issue_scat(jp)

    issue_idx(0, 0)
    issue_idx(1, 1)
    wait_idx(0, 0)
    issue_data(0, 0)
    pipe_iter(0, 0, first=True)
    for j in range(1, 6):
        pipe_iter(j, j)

    def group_body(t, _):
        k0 = t * 6
        for j in range(6):
            pipe_iter(k0 + j, j)
        return 0
    lax.fori_loop(1, 20, group_body, 0)

    for j in range(5):
        k = 120 + j
        pipe_iter(k, j, with_next=(k + 1 <= G - 1), with_next2=(k + 2 <= G - 1))
    wait_scat(4)

    plsc.subcore_barrier()

    for j in range(7):
        pltpu.sync_copy(acc_sh.at[pl.ds(r0 + j * C, C)], ybuf)
        pltpu.sync_copy(ybuf, acc_out.at[c, pl.ds(r0 + j * C, C)])
    pltpu.sync_copy(acc_sh.at[pl.ds(r0 + 7 * C, 64)], ybuf.at[pl.ds(0, 64)])
    pltpu.sync_copy(ybuf.at[pl.ds(0, 64)], acc_out.at[c, pl.ds(r0 + 7 * C, 64)])
    pltpu.sync_copy(deg_sh.at[pl.ds(r0, RPT)], dbuf)
    pltpu.sync_copy(dbuf, deg_out.at[pl.ds(c * N + r0, RPT)])

    @pl.when(s == NS - 1)
    def _copy_tail():
        pltpu.sync_copy(acc_sh.at[pl.ds(NS * RPT, 16)], ybuf.at[pl.ds(0, 16)])
        pltpu.sync_copy(ybuf.at[pl.ds(0, 16)], acc_out.at[c, pl.ds(NS * RPT, 16)])
        pltpu.sync_copy(deg_sh.at[pl.ds(NS * RPT, L)], dbuf.at[pl.ds(0, L)])
        pltpu.sync_copy(dbuf.at[pl.ds(0, L)], deg_out.at[pl.ds(c * N + NS * RPT, L)])


_sc_scatter = functools.partial(
    pl.kernel,
    out_type=[
        jax.ShapeDtypeStruct((NC, N, CH), jnp.float32),
        jax.ShapeDtypeStruct((NC * N,), jnp.float32),
    ],
    mesh=plsc.VectorSubcoreMesh(core_axis_name="c", subcore_axis_name="s",
                                num_cores=NC, num_subcores=NS),
    scratch_types=[
        pltpu.VMEM_SHARED((N, CH), jnp.float32),
        pltpu.VMEM_SHARED((N,), jnp.float32),
        pltpu.VMEM((3, C), jnp.int32),
        pltpu.VMEM((3, C), jnp.int32),
        pltpu.VMEM((2, C, CH), jnp.float32),
        pltpu.VMEM((2, C, CH), jnp.float32),
        pltpu.VMEM((C,), jnp.float32),
        pltpu.VMEM((RPT,), jnp.float32),
        pltpu.SemaphoreType.DMA,
        pltpu.SemaphoreType.DMA,
        pltpu.SemaphoreType.DMA,
        pltpu.SemaphoreType.DMA,
        pltpu.SemaphoreType.DMA,
        pltpu.SemaphoreType.DMA,
        pltpu.SemaphoreType.DMA,
        pltpu.SemaphoreType.DMA,
        pltpu.SemaphoreType.DMA,
    ],
)(_sc_scatter_body)


def kernel(x, edge_index, edge_attr, W_node, W1, b1, W2, b2, Wr, br):
    ei = edge_index.astype(jnp.int32)
    row = ei[0]
    col = ei[1]
    b1r = b1.reshape(1, CH).astype(jnp.float32)
    b2r = b2.reshape(1, CH).astype(jnp.float32)
    brr = br.reshape(1, CH).astype(jnp.float32)

    y = pl.pallas_call(
        _node_linear_body,
        grid=(N // ROW_BLK,),
        in_specs=[
            pl.BlockSpec((ROW_BLK, CH), lambda i: (i, 0)),
            pl.BlockSpec((CH, CH), lambda i: (0, 0)),
        ],
        out_specs=pl.BlockSpec((ROW_BLK, CH), lambda i: (i, 0)),
        out_shape=jax.ShapeDtypeStruct((N, CH), jnp.float32),
    )(x, W_node)

    gate = pl.pallas_call(
        _gate_body,
        grid=(E // EDGE_BLK,),
        in_specs=[
            pl.BlockSpec((EDGE_BLK, ECH), lambda i: (i, 0)),
            pl.BlockSpec((ECH, CH), lambda i: (0, 0)),
            pl.BlockSpec((1, CH), lambda i: (0, 0)),
            pl.BlockSpec((CH, CH), lambda i: (0, 0)),
            pl.BlockSpec((1, CH), lambda i: (0, 0)),
        ],
        out_specs=pl.BlockSpec((EDGE_BLK, CH), lambda i: (i, 0)),
        out_shape=jax.ShapeDtypeStruct((E, CH), jnp.float32),
    )(edge_attr, W1, b1r, W2, b2r)

    acc_parts, deg_flat = _sc_scatter(row, col, y, gate)
    deg2 = deg_flat.reshape(NC, N)

    out = pl.pallas_call(
        _combine_body,
        grid=(pl.cdiv(N, CROW),),
        in_specs=[
            pl.BlockSpec((CROW, CH), lambda i: (i, 0)),
            pl.BlockSpec((CROW, CH), lambda i: (i, 0)),
            pl.BlockSpec((1, CROW), lambda i: (0, i)),
            pl.BlockSpec((1, CROW), lambda i: (0, i)),
            pl.BlockSpec((CROW, CH), lambda i: (i, 0)),
            pl.BlockSpec((CROW, CH), lambda i: (i, 0)),
            pl.BlockSpec((CH, CH), lambda i: (0, 0)),
            pl.BlockSpec((1, CH), lambda i: (0, 0)),
            pl.BlockSpec((1, CH), lambda i: (0, 0)),
            pl.BlockSpec((CH, CH), lambda i: (0, 0)),
            pl.BlockSpec((1, CH), lambda i: (0, 0)),
        ],
        out_specs=pl.BlockSpec((CROW, CH), lambda i: (i, 0)),
        out_shape=jax.ShapeDtypeStruct((N, CH), jnp.float32),
    )(acc_parts[0], acc_parts[1], deg2[0:1], deg2[1:2], y, x,
      Wr, brr, b1r, W2, b2r)

    return out
